# baseline (device time: 22204 ns/iter reference)
import jax
import jax.numpy as jnp
from jax import lax
from jax.experimental import pallas as pl
from jax.experimental.pallas import tpu as pltpu

N_DEV = 4
N_TOK = 512
D_IN = 256
D_OUT = 512
N_EXP = 8
EXP_PER = N_EXP // N_DEV
CHUNK = N_TOK // N_DEV


def kernel(x, router_W, route_idx, expert_W):
    def body(x_ref, rw_ref, idx_ref, ew_ref, out_ref,
             acc_ref, comm_ref, send_sems, recv_sems):
        my = lax.axis_index("i")
        left = jnp.mod(my - 1, N_DEV)
        right = jnp.mod(my + 1, N_DEV)

        barrier_sem = pltpu.get_barrier_semaphore()
        for nbr in [left, right]:
            pl.semaphore_signal(
                barrier_sem, inc=1,
                device_id=(nbr,), device_id_type=pl.DeviceIdType.MESH,
            )
        pl.semaphore_wait(barrier_sem, 2)

        xv = x_ref[:, :]
        scores = jnp.dot(xv, rw_ref[:, :], preferred_element_type=jnp.float32)
        m = jnp.max(scores, axis=-1, keepdims=True)
        p = jnp.exp(scores - m)
        p = p / jnp.sum(p, axis=-1, keepdims=True)

        idx = idx_ref[:, :]
        cols = lax.broadcasted_iota(jnp.int32, (N_TOK, N_EXP), 1)
        sel0 = idx[:, 0:1] == cols
        sel1 = idx[:, 1:2] == cols
        g0 = jnp.sum(jnp.where(sel0, p, 0.0), axis=1, keepdims=True)
        g1 = jnp.sum(jnp.where(sel1, p, 0.0), axis=1, keepdims=True)
        wfull = jnp.where(sel0 | sel1, p, 0.0) / (g0 + g1)

        e_base = my * EXP_PER
        w0 = jnp.sum(jnp.where(cols == e_base, wfull, 0.0), axis=1, keepdims=True)
        w1 = jnp.sum(jnp.where(cols == e_base + 1, wfull, 0.0), axis=1, keepdims=True)
        partial = (
            jnp.dot(w0 * xv, ew_ref[0], preferred_element_type=jnp.float32)
            + jnp.dot(w1 * xv, ew_ref[1], preferred_element_type=jnp.float32)
        )
        acc_ref[:, :] = partial

        for h in range(N_DEV - 1):
            slot = h % 2
            c_send = jnp.mod(my - h - 1, N_DEV)
            c_recv = jnp.mod(my - h - 2, N_DEV)
            rdma = pltpu.make_async_remote_copy(
                src_ref=acc_ref.at[pl.ds(c_send * CHUNK, CHUNK), :],
                dst_ref=comm_ref.at[slot],
                send_sem=send_sems.at[slot],
                recv_sem=recv_sems.at[slot],
                device_id=(right,),
                device_id_type=pl.DeviceIdType.MESH,
            )
            rdma.start()
            rdma.wait()
            acc_ref[pl.ds(c_recv * CHUNK, CHUNK), :] = (
                acc_ref[pl.ds(c_recv * CHUNK, CHUNK), :] + comm_ref[slot]
            )

        out_ref[:, :] = acc_ref[pl.ds(my * CHUNK, CHUNK), :]

    return pl.pallas_call(
        body,
        out_shape=jax.ShapeDtypeStruct((CHUNK, D_OUT), jnp.float32),
        in_specs=[
            pl.BlockSpec(memory_space=pltpu.VMEM),
            pl.BlockSpec(memory_space=pltpu.VMEM),
            pl.BlockSpec(memory_space=pltpu.VMEM),
            pl.BlockSpec(memory_space=pltpu.VMEM),
        ],
        out_specs=pl.BlockSpec(memory_space=pltpu.VMEM),
        scratch_shapes=[
            pltpu.VMEM((N_TOK, D_OUT), jnp.float32),
            pltpu.VMEM((2, CHUNK, D_OUT), jnp.float32),
            pltpu.SemaphoreType.DMA((2,)),
            pltpu.SemaphoreType.DMA((2,)),
        ],
        compiler_params=pltpu.CompilerParams(collective_id=0),
    )(x, router_W, route_idx, expert_W)


# device time: 15407 ns/iter; 1.4412x vs baseline; 1.4412x over previous
import jax
import jax.numpy as jnp
from jax import lax
from jax.experimental import pallas as pl
from jax.experimental.pallas import tpu as pltpu

N_DEV = 4
N_TOK = 512
D_IN = 256
D_OUT = 512
N_EXP = 8
EXP_PER = N_EXP // N_DEV
CHUNK = N_TOK // N_DEV


def kernel(x, router_W, route_idx, expert_W):
    def body(x_ref, rw_ref, idx_ref, ew_ref, out_ref,
             xw_ref, send_buf, comm_ref, send_sems, recv_sems):
        my = lax.axis_index("i")

        barrier_sem = pltpu.get_barrier_semaphore()
        for k in range(1, N_DEV):
            pl.semaphore_signal(
                barrier_sem, inc=1,
                device_id=(jnp.mod(my + k, N_DEV),),
                device_id_type=pl.DeviceIdType.MESH,
            )

        xv = x_ref[:, :]
        scores = jnp.dot(xv, rw_ref[:, :], preferred_element_type=jnp.float32)
        m = jnp.max(scores, axis=-1, keepdims=True)
        p = jnp.exp(scores - m)
        p = p / jnp.sum(p, axis=-1, keepdims=True)

        idx = idx_ref[:, :]
        cols = lax.broadcasted_iota(jnp.int32, (N_TOK, N_EXP), 1)
        sel0 = idx[:, 0:1] == cols
        sel1 = idx[:, 1:2] == cols
        g0 = jnp.sum(jnp.where(sel0, p, 0.0), axis=1, keepdims=True)
        g1 = jnp.sum(jnp.where(sel1, p, 0.0), axis=1, keepdims=True)
        wfull = jnp.where(sel0 | sel1, p, 0.0) / (g0 + g1)

        e_base = my * EXP_PER
        w0 = jnp.sum(jnp.where(cols == e_base, wfull, 0.0), axis=1, keepdims=True)
        w1 = jnp.sum(jnp.where(cols == e_base + 1, wfull, 0.0), axis=1, keepdims=True)
        xw_ref[0, :, :] = w0 * xv
        xw_ref[1, :, :] = w1 * xv
        W0 = ew_ref[0]
        W1 = ew_ref[1]

        pl.semaphore_wait(barrier_sem, N_DEV - 1)

        sends = []
        for k in range(1, N_DEV):
            dst = jnp.mod(my + k, N_DEV)
            rows = pl.ds(dst * CHUNK, CHUNK)
            chunk = (
                jnp.dot(xw_ref[0, rows, :], W0, preferred_element_type=jnp.float32)
                + jnp.dot(xw_ref[1, rows, :], W1, preferred_element_type=jnp.float32)
            )
            send_buf[rows, :] = chunk
            rdma = pltpu.make_async_remote_copy(
                src_ref=send_buf.at[rows, :],
                dst_ref=comm_ref.at[pl.ds(my * CHUNK, CHUNK), :],
                send_sem=send_sems.at[k - 1],
                recv_sem=recv_sems.at[my],
                device_id=(dst,),
                device_id_type=pl.DeviceIdType.MESH,
            )
            rdma.start()
            sends.append(rdma)

        my_rows = pl.ds(my * CHUNK, CHUNK)
        own = (
            jnp.dot(xw_ref[0, my_rows, :], W0, preferred_element_type=jnp.float32)
            + jnp.dot(xw_ref[1, my_rows, :], W1, preferred_element_type=jnp.float32)
        )

        acc = own
        for k in range(1, N_DEV):
            src = jnp.mod(my - k, N_DEV)
            recv = pltpu.make_async_remote_copy(
                src_ref=send_buf.at[pl.ds(0, CHUNK), :],
                dst_ref=comm_ref.at[pl.ds(src * CHUNK, CHUNK), :],
                send_sem=send_sems.at[k - 1],
                recv_sem=recv_sems.at[src],
                device_id=(src,),
                device_id_type=pl.DeviceIdType.MESH,
            )
            recv.wait_recv()
            acc = acc + comm_ref[pl.ds(src * CHUNK, CHUNK), :]
        out_ref[:, :] = acc

        for rdma in sends:
            rdma.wait_send()

    return pl.pallas_call(
        body,
        out_shape=jax.ShapeDtypeStruct((CHUNK, D_OUT), jnp.float32),
        in_specs=[
            pl.BlockSpec(memory_space=pltpu.VMEM),
            pl.BlockSpec(memory_space=pltpu.VMEM),
            pl.BlockSpec(memory_space=pltpu.VMEM),
            pl.BlockSpec(memory_space=pltpu.VMEM),
        ],
        out_specs=pl.BlockSpec(memory_space=pltpu.VMEM),
        scratch_shapes=[
            pltpu.VMEM((2, N_TOK, D_IN), jnp.float32),
            pltpu.VMEM((N_TOK, D_OUT), jnp.float32),
            pltpu.VMEM((N_TOK, D_OUT), jnp.float32),
            pltpu.SemaphoreType.DMA((N_DEV - 1,)),
            pltpu.SemaphoreType.DMA((N_DEV,)),
        ],
        compiler_params=pltpu.CompilerParams(collective_id=0),
    )(x, router_W, route_idx, expert_W)
